# Initial kernel scaffold; baseline (speedup 1.0000x reference)
import functools

import jax
import jax.numpy as jnp
from jax import lax
from jax.experimental import pallas as pl
from jax.experimental.pallas import tpu as pltpu
from jax.experimental.pallas import tpu_sc as plsc

VOCAB = 1000000
EMB = 32
BATCH = 4096
SEQ = 200

_B = BATCH * SEQ

_info = plsc.get_sparse_core_info()
_NC, _NS = _info.num_cores, _info.num_subcores
_NW = _NC * _NS
_BPW = _B // _NW
_CHUNK = 1280
_NCHUNK = _BPW // _CHUNK
_SUB = 128
_NSUB = _CHUNK // _SUB


def _emb_body(x_hbm, table_hbm, out_hbm, idx_v, rows_v, sem):
    wid = lax.axis_index("s") * _NC + lax.axis_index("c")
    base = wid * _BPW

    def chunk_body(i, carry):
        off = base + i * _CHUNK
        pltpu.sync_copy(x_hbm.at[pl.ds(off, _CHUNK)], idx_v)
        copies = [
            pltpu.async_copy(
                table_hbm.at[idx_v.at[pl.ds(j * _SUB, _SUB)]],
                rows_v.at[pl.ds(j * _SUB, _SUB)],
                sem,
            )
            for j in range(_NSUB)
        ]
        for c in copies:
            c.wait()
        pltpu.sync_copy(rows_v, out_hbm.at[pl.ds(off, _CHUNK)])
        return carry

    lax.fori_loop(0, _NCHUNK, chunk_body, 0)


@functools.partial(
    pl.kernel,
    mesh=plsc.VectorSubcoreMesh(core_axis_name="c", subcore_axis_name="s"),
    out_type=jax.ShapeDtypeStruct((_B, EMB), jnp.float32),
    scratch_types=[
        pltpu.VMEM((_CHUNK,), jnp.int32),
        pltpu.VMEM((_CHUNK, EMB), jnp.float32),
        pltpu.SemaphoreType.DMA,
    ],
)
def _emb_lookup(x_hbm, table_hbm, out_hbm, idx_v, rows_v, sem):
    _emb_body(x_hbm, table_hbm, out_hbm, idx_v, rows_v, sem)


def kernel(x, table):
    flat = _emb_lookup(x.reshape(_B), table)
    return flat.reshape(BATCH, SEQ, EMB)


# speedup vs baseline: 1.4689x; 1.4689x over previous
import functools

import jax
import jax.numpy as jnp
from jax import lax
from jax.experimental import pallas as pl
from jax.experimental.pallas import tpu as pltpu
from jax.experimental.pallas import tpu_sc as plsc

VOCAB = 1000000
EMB = 32
BATCH = 4096
SEQ = 200

_B = BATCH * SEQ

_NC, _NS = 2, 16
_NW = _NC * _NS
_BPW = _B // _NW
_CHUNK = 1280
_NCHUNK = _BPW // _CHUNK
_SUB = 128
_NSUB = _CHUNK // _SUB


def _emb_body(x_hbm, table_hbm, out_hbm, idx_v, rows_v, sem):
    wid = lax.axis_index("s") * _NC + lax.axis_index("c")
    base = wid * _BPW

    def chunk_body(i, carry):
        off = base + i * _CHUNK
        pltpu.sync_copy(x_hbm.at[pl.ds(off, _CHUNK)], idx_v)
        copies = [
            pltpu.async_copy(
                table_hbm.at[idx_v.at[pl.ds(j * _SUB, _SUB)]],
                rows_v.at[pl.ds(j * _SUB, _SUB)],
                sem,
            )
            for j in range(_NSUB)
        ]
        for c in copies:
            c.wait()
        pltpu.sync_copy(rows_v, out_hbm.at[pl.ds(off, _CHUNK)])
        return carry

    lax.fori_loop(0, _NCHUNK, chunk_body, 0)


@functools.cache
def _build():
    return pl.kernel(
        _emb_body,
        mesh=plsc.VectorSubcoreMesh(core_axis_name="c", subcore_axis_name="s"),
        out_type=jax.ShapeDtypeStruct((_B, EMB), jnp.float32),
        scratch_types=[
            pltpu.VMEM((_CHUNK,), jnp.int32),
            pltpu.VMEM((_CHUNK, EMB), jnp.float32),
            pltpu.SemaphoreType.DMA,
        ],
        compiler_params=pltpu.CompilerParams(use_tc_tiling_on_sc=False),
    )


def kernel(x, table):
    flat = _build()(x.reshape(_B), table)
    return flat.reshape(BATCH, SEQ, EMB)


# speedup vs baseline: 1.4932x; 1.0165x over previous
import functools

import jax
import jax.numpy as jnp
from jax import lax
from jax.experimental import pallas as pl
from jax.experimental.pallas import tpu as pltpu
from jax.experimental.pallas import tpu_sc as plsc

VOCAB = 1000000
EMB = 32
BATCH = 4096
SEQ = 200

_B = BATCH * SEQ
_NC, _NS = 2, 16
_NW = _NC * _NS
_BPW = _B // _NW
_CHUNK = 1280
_NCHUNK = _BPW // _CHUNK
_SUB = 128
_NSUB = _CHUNK // _SUB
_NBUF = 2
_NSTEP = _NCHUNK // _NBUF


def _emb_body(x_hbm, table_hbm, out_hbm,
              idx0, idx1, rows0, rows1,
              isem0, isem1, gsem, osem0, osem1):
    wid = lax.axis_index("s") * _NC + lax.axis_index("c")
    base = wid * _BPW
    idx = (idx0, idx1)
    rows = (rows0, rows1)
    isem = (isem0, isem1)
    osem = (osem0, osem1)

    def idx_copy(b, g):
        off = base + g * _CHUNK
        return pltpu.make_async_copy(
            x_hbm.at[pl.ds(off, _CHUNK)], idx[b], isem[b])

    def out_copy(b, g):
        off = base + g * _CHUNK
        return pltpu.make_async_copy(
            rows[b], out_hbm.at[pl.ds(off, _CHUNK)], osem[b])

    for b in range(_NBUF):
        idx_copy(b, b).start()

    def step(s, carry):
        for b in range(_NBUF):
            g = s * _NBUF + b
            idx_copy(b, g).wait()

            @pl.when(s > 0)
            def _():
                out_copy(b, g).wait()

            gathers = [
                pltpu.make_async_copy(
                    table_hbm.at[idx[b].at[pl.ds(j * _SUB, _SUB)]],
                    rows[b].at[pl.ds(j * _SUB, _SUB)],
                    gsem,
                )
                for j in range(_NSUB)
            ]
            for c in gathers:
                c.start()
            for c in gathers:
                c.wait()
            gnext = jnp.minimum(g + _NBUF, _NCHUNK - 1)
            idx_copy(b, gnext).start()
            out_copy(b, g).start()
        return carry

    lax.fori_loop(0, _NSTEP, step, 0)

    for b in range(_NBUF):
        idx_copy(b, 0).wait()
        out_copy(b, 0).wait()


@functools.cache
def _build():
    return pl.kernel(
        _emb_body,
        mesh=plsc.VectorSubcoreMesh(core_axis_name="c", subcore_axis_name="s"),
        out_type=jax.ShapeDtypeStruct((_B, EMB), jnp.float32),
        scratch_types=[
            pltpu.VMEM((_CHUNK,), jnp.int32),
            pltpu.VMEM((_CHUNK,), jnp.int32),
            pltpu.VMEM((_CHUNK, EMB), jnp.float32),
            pltpu.VMEM((_CHUNK, EMB), jnp.float32),
            pltpu.SemaphoreType.DMA,
            pltpu.SemaphoreType.DMA,
            pltpu.SemaphoreType.DMA,
            pltpu.SemaphoreType.DMA,
            pltpu.SemaphoreType.DMA,
        ],
        compiler_params=pltpu.CompilerParams(use_tc_tiling_on_sc=False),
    )


def kernel(x, table):
    flat = _build()(x.reshape(_B), table)
    return flat.reshape(BATCH, SEQ, EMB)
